# baseline (speedup 1.0000x reference)
import jax
import jax.numpy as jnp
from jax.experimental import pallas as pl
from jax.experimental.pallas import tpu as pltpu

B, N, C = 8, 2048, 128
BLK = 1024
NBLK = N // BLK
H = N // 2
EPS = 1e-5


def _stats_init_and_accum(r, i, sum_ref, sq_ref):
    ps = jnp.sum(r, axis=0, keepdims=True)
    pq = jnp.sum(r * r, axis=0, keepdims=True)

    @pl.when(i == 0)
    def _():
        sum_ref[0] = ps
        sq_ref[0] = pq

    @pl.when(i > 0)
    def _():
        sum_ref[0] += ps
        sq_ref[0] += pq


def _compute_z(h_ref, w_ref, bias_ref, z_ref):
    z = (
        jnp.dot(h_ref[0], w_ref[...], preferred_element_type=jnp.float32)
        + bias_ref[...]
    )
    z_ref[...] = z.astype(jnp.bfloat16)


def _layer0_body(
    adjl_ref, adjr_ref, h_ref, w_ref, bias_ref,
    r_ref, sum_ref, sq_ref, abfl_ref, abfr_ref, z_ref,
):
    i = pl.program_id(1)

    @pl.when(i == 0)
    def _():
        _compute_z(h_ref, w_ref, bias_ref, z_ref)

    rloc = jax.lax.broadcasted_iota(jnp.int32, (BLK, H), 0)
    cloc = jax.lax.broadcasted_iota(jnp.int32, (BLK, H), 1)
    eye = rloc == cloc
    al = jnp.where(eye & (i == 0), 1.0, adjl_ref[0]).astype(jnp.bfloat16)
    ar = jnp.where(eye & (i == 1), 1.0, adjr_ref[0]).astype(jnp.bfloat16)
    abfl_ref[0] = al
    abfr_ref[0] = ar

    m = jnp.dot(al, z_ref[:H], preferred_element_type=jnp.float32) + jnp.dot(
        ar, z_ref[H:], preferred_element_type=jnp.float32
    )
    r = jnp.maximum(m, 0.0)
    r_ref[0] = r
    _stats_init_and_accum(r, i, sum_ref, sq_ref)


def _layer_body(
    adjl_ref, adjr_ref, h_ref, w_ref, bias_ref, r_ref, sum_ref, sq_ref, z_ref
):
    i = pl.program_id(1)

    @pl.when(i == 0)
    def _():
        _compute_z(h_ref, w_ref, bias_ref, z_ref)

    m = jnp.dot(
        adjl_ref[0], z_ref[:H], preferred_element_type=jnp.float32
    ) + jnp.dot(adjr_ref[0], z_ref[H:], preferred_element_type=jnp.float32)
    r = jnp.maximum(m, 0.0)
    r_ref[0] = r
    _stats_init_and_accum(r, i, sum_ref, sq_ref)


_ADJL_SPEC = pl.BlockSpec((1, BLK, H), lambda b, i: (b, i, 0))
_ADJR_SPEC = pl.BlockSpec((1, BLK, H), lambda b, i: (b, i, 1))
_AHALF_SPEC = pl.BlockSpec((1, BLK, H), lambda b, i: (b, i, 0))
_H_SPEC = pl.BlockSpec((1, N, C), lambda b, i: (b, 0, 0))
_W_SPEC = pl.BlockSpec((C, C), lambda b, i: (0, 0))
_BIAS_SPEC = pl.BlockSpec((1, C), lambda b, i: (0, 0))
_RBLK_SPEC = pl.BlockSpec((1, BLK, C), lambda b, i: (b, i, 0))
_STAT_SPEC = pl.BlockSpec((1, 1, C), lambda b, i: (b, 0, 0))
_STAT_SHAPE = jax.ShapeDtypeStruct((B, 1, C), jnp.float32)
_BNC_SHAPE = jax.ShapeDtypeStruct((B, N, C), jnp.float32)
_AHALF_SHAPE = jax.ShapeDtypeStruct((B, N, H), jnp.bfloat16)
_PARAMS = pltpu.CompilerParams(dimension_semantics=("parallel", "arbitrary"))


def _layer0(adj, h, w_eff, b_eff):
    return pl.pallas_call(
        _layer0_body,
        grid=(B, NBLK),
        in_specs=[_ADJL_SPEC, _ADJR_SPEC, _H_SPEC, _W_SPEC, _BIAS_SPEC],
        out_specs=[_RBLK_SPEC, _STAT_SPEC, _STAT_SPEC, _AHALF_SPEC, _AHALF_SPEC],
        out_shape=[_BNC_SHAPE, _STAT_SHAPE, _STAT_SHAPE, _AHALF_SHAPE, _AHALF_SHAPE],
        scratch_shapes=[pltpu.VMEM((N, C), jnp.bfloat16)],
        compiler_params=_PARAMS,
    )(adj, adj, h, w_eff, b_eff.reshape(1, C))


def _layer(abfl, abfr, h, w_eff, b_eff):
    return pl.pallas_call(
        _layer_body,
        grid=(B, NBLK),
        in_specs=[_AHALF_SPEC, _AHALF_SPEC, _H_SPEC, _W_SPEC, _BIAS_SPEC],
        out_specs=[_RBLK_SPEC, _STAT_SPEC, _STAT_SPEC],
        out_shape=[_BNC_SHAPE, _STAT_SHAPE, _STAT_SHAPE],
        scratch_shapes=[pltpu.VMEM((N, C), jnp.bfloat16)],
        compiler_params=_PARAMS,
    )(abfl, abfr, h, w_eff, b_eff.reshape(1, C))


def _affine_body(r_ref, s_ref, t_ref, o_ref):
    o_ref[...] = r_ref[...] * s_ref[...] + t_ref[...]


def _final_affine(r, s, t):
    rf = r.reshape(B * N, C)
    out = pl.pallas_call(
        _affine_body,
        grid=(B * N // 2048,),
        in_specs=[
            pl.BlockSpec((2048, C), lambda i: (i, 0)),
            pl.BlockSpec((1, C), lambda i: (0, 0)),
            pl.BlockSpec((1, C), lambda i: (0, 0)),
        ],
        out_specs=pl.BlockSpec((2048, C), lambda i: (i, 0)),
        out_shape=jax.ShapeDtypeStruct((B * N, C), jnp.float32),
    )(rf, s.reshape(1, C), t.reshape(1, C))
    return out.reshape(B, N, C)


def kernel(x, adj, W0, b0, gamma0, beta0, W1, b1, gamma1, beta1, W2, b2, gamma2, beta2):
    Ws = [W0, W1, W2]
    bs = [b0, b1, b2]
    gammas = [gamma0, gamma1, gamma2]
    betas = [beta0, beta1, beta2]

    h = x
    s = t = abfl = abfr = None
    cnt = float(B * N)
    for l in range(3):
        if l == 0:
            h, sm, sq, abfl, abfr = _layer0(adj, h, Ws[0], bs[0])
        else:
            w_eff = s[:, None] * Ws[l]
            b_eff = t @ Ws[l] + bs[l]
            h, sm, sq = _layer(abfl, abfr, h, w_eff, b_eff)
        mean = jnp.sum(sm, axis=(0, 1)) / cnt
        var = jnp.sum(sq, axis=(0, 1)) / cnt - mean * mean
        s = gammas[l] * jax.lax.rsqrt(var + EPS)
        t = betas[l] - mean * s
    return _final_affine(h, s, t)


# speedup vs baseline: 1.0491x; 1.0491x over previous
import jax
import jax.numpy as jnp
from jax.experimental import pallas as pl
from jax.experimental.pallas import tpu as pltpu

B, N, C = 8, 2048, 128
BLK = 1024
NBLK = N // BLK
EPS = 1e-5


def _stats_init_and_accum(r, i, sum_ref, sq_ref):
    ps = jnp.sum(r, axis=0, keepdims=True)
    pq = jnp.sum(r * r, axis=0, keepdims=True)

    @pl.when(i == 0)
    def _():
        sum_ref[0] = ps
        sq_ref[0] = pq

    @pl.when(i > 0)
    def _():
        sum_ref[0] += ps
        sq_ref[0] += pq


def _layer0_body(adj_ref, x_ref, w_ref, bias_ref, r_ref, sum_ref, sq_ref, abf_ref, z_ref):
    i = pl.program_id(1)

    @pl.when(i == 0)
    def _():
        z = (
            jnp.dot(x_ref[0], w_ref[...], preferred_element_type=jnp.float32)
            + bias_ref[...]
        )
        z_ref[...] = z.astype(jnp.bfloat16)

    rows = i * BLK + jax.lax.broadcasted_iota(jnp.int32, (BLK, N), 0)
    cols = jax.lax.broadcasted_iota(jnp.int32, (BLK, N), 1)
    abf = jnp.where(rows == cols, 1.0, adj_ref[0]).astype(jnp.bfloat16)
    abf_ref[0] = abf

    m = jnp.dot(abf, z_ref[...], preferred_element_type=jnp.float32)
    r = jnp.maximum(m, 0.0)
    r_ref[0] = r.astype(jnp.bfloat16)
    _stats_init_and_accum(r, i, sum_ref, sq_ref)


def _layer_body(adj_ref, h_ref, w_ref, bias_ref, r_ref, sum_ref, sq_ref, z_ref):
    i = pl.program_id(1)

    @pl.when(i == 0)
    def _():
        z = (
            jnp.dot(
                h_ref[0],
                w_ref[...].astype(jnp.bfloat16),
                preferred_element_type=jnp.float32,
            )
            + bias_ref[...]
        )
        z_ref[...] = z.astype(jnp.bfloat16)

    m = jnp.dot(adj_ref[0], z_ref[...], preferred_element_type=jnp.float32)
    r = jnp.maximum(m, 0.0)
    r_ref[0] = r.astype(jnp.bfloat16)
    _stats_init_and_accum(r, i, sum_ref, sq_ref)


_ADJ_SPEC = pl.BlockSpec((1, BLK, N), lambda b, i: (b, i, 0))
_H_SPEC = pl.BlockSpec((1, N, C), lambda b, i: (b, 0, 0))
_W_SPEC = pl.BlockSpec((C, C), lambda b, i: (0, 0))
_BIAS_SPEC = pl.BlockSpec((1, C), lambda b, i: (0, 0))
_RBLK_SPEC = pl.BlockSpec((1, BLK, C), lambda b, i: (b, i, 0))
_STAT_SPEC = pl.BlockSpec((1, 1, C), lambda b, i: (b, 0, 0))
_STAT_SHAPE = jax.ShapeDtypeStruct((B, 1, C), jnp.float32)
_BNC_BF_SHAPE = jax.ShapeDtypeStruct((B, N, C), jnp.bfloat16)
_PARAMS = pltpu.CompilerParams(dimension_semantics=("parallel", "arbitrary"))


def _layer0(adj, x, w_eff, b_eff):
    return pl.pallas_call(
        _layer0_body,
        grid=(B, NBLK),
        in_specs=[_ADJ_SPEC, _H_SPEC, _W_SPEC, _BIAS_SPEC],
        out_specs=[_RBLK_SPEC, _STAT_SPEC, _STAT_SPEC, _ADJ_SPEC],
        out_shape=[
            _BNC_BF_SHAPE,
            _STAT_SHAPE,
            _STAT_SHAPE,
            jax.ShapeDtypeStruct((B, N, N), jnp.bfloat16),
        ],
        scratch_shapes=[pltpu.VMEM((N, C), jnp.bfloat16)],
        compiler_params=_PARAMS,
    )(adj, x, w_eff, b_eff.reshape(1, C))


def _layer(adj_bf, h, w_eff, b_eff):
    return pl.pallas_call(
        _layer_body,
        grid=(B, NBLK),
        in_specs=[_ADJ_SPEC, _H_SPEC, _W_SPEC, _BIAS_SPEC],
        out_specs=[_RBLK_SPEC, _STAT_SPEC, _STAT_SPEC],
        out_shape=[_BNC_BF_SHAPE, _STAT_SHAPE, _STAT_SHAPE],
        scratch_shapes=[pltpu.VMEM((N, C), jnp.bfloat16)],
        compiler_params=_PARAMS,
    )(adj_bf, h, w_eff, b_eff.reshape(1, C))


def _affine_body(r_ref, s_ref, t_ref, o_ref):
    o_ref[...] = r_ref[...].astype(jnp.float32) * s_ref[...] + t_ref[...]


def _final_affine(r, s, t):
    rf = r.reshape(B * N, C)
    out = pl.pallas_call(
        _affine_body,
        grid=(B * N // 2048,),
        in_specs=[
            pl.BlockSpec((2048, C), lambda i: (i, 0)),
            pl.BlockSpec((1, C), lambda i: (0, 0)),
            pl.BlockSpec((1, C), lambda i: (0, 0)),
        ],
        out_specs=pl.BlockSpec((2048, C), lambda i: (i, 0)),
        out_shape=jax.ShapeDtypeStruct((B * N, C), jnp.float32),
    )(rf, s.reshape(1, C), t.reshape(1, C))
    return out.reshape(B, N, C)


def kernel(x, adj, W0, b0, gamma0, beta0, W1, b1, gamma1, beta1, W2, b2, gamma2, beta2):
    Ws = [W0, W1, W2]
    bs = [b0, b1, b2]
    gammas = [gamma0, gamma1, gamma2]
    betas = [beta0, beta1, beta2]

    h = x
    s = t = adj_bf = None
    cnt = float(B * N)
    for l in range(3):
        if l == 0:
            h, sm, sq, adj_bf = _layer0(adj, h, Ws[0], bs[0])
        else:
            w_eff = s[:, None] * Ws[l]
            b_eff = t @ Ws[l] + bs[l]
            h, sm, sq = _layer(adj_bf, h, w_eff, b_eff)
        mean = jnp.sum(sm, axis=(0, 1)) / cnt
        var = jnp.sum(sq, axis=(0, 1)) / cnt - mean * mean
        s = gammas[l] * jax.lax.rsqrt(var + EPS)
        t = betas[l] - mean * s
    return _final_affine(h, s, t)


# speedup vs baseline: 1.1282x; 1.0754x over previous
import jax
import jax.numpy as jnp
from jax.experimental import pallas as pl
from jax.experimental.pallas import tpu as pltpu

B, N, C = 8, 2048, 128
BLK = 1024
NBLK = N // BLK
EPS = 1e-5


def _stats_init_and_accum(r, i, sum_ref, sq_ref):
    ps = jnp.sum(r, axis=0, keepdims=True)
    pq = jnp.sum(r * r, axis=0, keepdims=True)

    @pl.when(i == 0)
    def _():
        sum_ref[0] = ps
        sq_ref[0] = pq

    @pl.when(i > 0)
    def _():
        sum_ref[0] += ps
        sq_ref[0] += pq


def _layer0_body(adj_ref, x_ref, w_ref, bias_ref, r_ref, sum_ref, sq_ref, abf_ref, z_ref):
    i = pl.program_id(1)

    @pl.when(i == 0)
    def _():
        z = (
            jnp.dot(x_ref[0], w_ref[...], preferred_element_type=jnp.float32)
            + bias_ref[...]
        )
        z_ref[...] = z.astype(jnp.bfloat16)

    rows = i * BLK + jax.lax.broadcasted_iota(jnp.int32, (BLK, N), 0)
    cols = jax.lax.broadcasted_iota(jnp.int32, (BLK, N), 1)
    abf = jnp.where(rows == cols, 1.0, adj_ref[0]).astype(jnp.bfloat16)
    abf_ref[0] = abf

    m = jnp.dot(abf, z_ref[...], preferred_element_type=jnp.float32)
    r = jnp.maximum(m, 0.0)
    r_ref[0] = r.astype(jnp.bfloat16)
    _stats_init_and_accum(r, i, sum_ref, sq_ref)


def _layer_body(adj_ref, h_ref, w_ref, bias_ref, r_ref, sum_ref, sq_ref):
    z = (
        jnp.dot(
            h_ref[0],
            w_ref[...].astype(jnp.bfloat16),
            preferred_element_type=jnp.float32,
        )
        + bias_ref[...]
    ).astype(jnp.bfloat16)

    m = jnp.dot(adj_ref[0], z, preferred_element_type=jnp.float32)
    r = jnp.maximum(m, 0.0)
    r_ref[0] = r.astype(jnp.bfloat16)
    sum_ref[0] = jnp.sum(r, axis=0, keepdims=True)
    sq_ref[0] = jnp.sum(r * r, axis=0, keepdims=True)


_ADJ_SPEC = pl.BlockSpec((1, BLK, N), lambda b, i: (b, i, 0))
_H_SPEC = pl.BlockSpec((1, N, C), lambda b, i: (b, 0, 0))
_W_SPEC = pl.BlockSpec((C, C), lambda b, i: (0, 0))
_BIAS_SPEC = pl.BlockSpec((1, C), lambda b, i: (0, 0))
_RBLK_SPEC = pl.BlockSpec((1, BLK, C), lambda b, i: (b, i, 0))
_STAT_SPEC = pl.BlockSpec((1, 1, C), lambda b, i: (b, 0, 0))
_STAT_SHAPE = jax.ShapeDtypeStruct((B, 1, C), jnp.float32)
_BNC_BF_SHAPE = jax.ShapeDtypeStruct((B, N, C), jnp.bfloat16)
_PARAMS = pltpu.CompilerParams(dimension_semantics=("parallel", "arbitrary"))


def _layer0(adj, x, w_eff, b_eff):
    return pl.pallas_call(
        _layer0_body,
        grid=(B, NBLK),
        in_specs=[_ADJ_SPEC, _H_SPEC, _W_SPEC, _BIAS_SPEC],
        out_specs=[_RBLK_SPEC, _STAT_SPEC, _STAT_SPEC, _ADJ_SPEC],
        out_shape=[
            _BNC_BF_SHAPE,
            _STAT_SHAPE,
            _STAT_SHAPE,
            jax.ShapeDtypeStruct((B, N, N), jnp.bfloat16),
        ],
        scratch_shapes=[pltpu.VMEM((N, C), jnp.bfloat16)],
        compiler_params=_PARAMS,
    )(adj, x, w_eff, b_eff.reshape(1, C))


_ADJ2_SPEC = pl.BlockSpec((1, N, N), lambda b: (b, 0, 0))
_H2_SPEC = pl.BlockSpec((1, N, C), lambda b: (b, 0, 0))
_W2_SPEC = pl.BlockSpec((C, C), lambda b: (0, 0))
_BIAS2_SPEC = pl.BlockSpec((1, C), lambda b: (0, 0))
_R2_SPEC = pl.BlockSpec((1, N, C), lambda b: (b, 0, 0))
_STAT2_SPEC = pl.BlockSpec((1, 1, C), lambda b: (b, 0, 0))


def _layer(adj_bf, h, w_eff, b_eff):
    return pl.pallas_call(
        _layer_body,
        grid=(B,),
        in_specs=[_ADJ2_SPEC, _H2_SPEC, _W2_SPEC, _BIAS2_SPEC],
        out_specs=[_R2_SPEC, _STAT2_SPEC, _STAT2_SPEC],
        out_shape=[_BNC_BF_SHAPE, _STAT_SHAPE, _STAT_SHAPE],
        compiler_params=pltpu.CompilerParams(dimension_semantics=("parallel",)),
    )(adj_bf, h, w_eff, b_eff.reshape(1, C))


def _affine_body(r_ref, s_ref, t_ref, o_ref):
    o_ref[...] = r_ref[...].astype(jnp.float32) * s_ref[...] + t_ref[...]


def _final_affine(r, s, t):
    rf = r.reshape(B * N, C)
    out = pl.pallas_call(
        _affine_body,
        grid=(B * N // 2048,),
        in_specs=[
            pl.BlockSpec((2048, C), lambda i: (i, 0)),
            pl.BlockSpec((1, C), lambda i: (0, 0)),
            pl.BlockSpec((1, C), lambda i: (0, 0)),
        ],
        out_specs=pl.BlockSpec((2048, C), lambda i: (i, 0)),
        out_shape=jax.ShapeDtypeStruct((B * N, C), jnp.float32),
    )(rf, s.reshape(1, C), t.reshape(1, C))
    return out.reshape(B, N, C)


def kernel(x, adj, W0, b0, gamma0, beta0, W1, b1, gamma1, beta1, W2, b2, gamma2, beta2):
    Ws = [W0, W1, W2]
    bs = [b0, b1, b2]
    gammas = [gamma0, gamma1, gamma2]
    betas = [beta0, beta1, beta2]

    h = x
    s = t = adj_bf = None
    cnt = float(B * N)
    for l in range(3):
        if l == 0:
            h, sm, sq, adj_bf = _layer0(adj, h, Ws[0], bs[0])
        else:
            w_eff = s[:, None] * Ws[l]
            b_eff = t @ Ws[l] + bs[l]
            h, sm, sq = _layer(adj_bf, h, w_eff, b_eff)
        mean = jnp.sum(sm, axis=(0, 1)) / cnt
        var = jnp.sum(sq, axis=(0, 1)) / cnt - mean * mean
        s = gammas[l] * jax.lax.rsqrt(var + EPS)
        t = betas[l] - mean * s
    return _final_affine(h, s, t)
